# baseline (speedup 1.0000x reference)
import functools

import jax
import jax.numpy as jnp
from jax import lax
from jax.experimental import pallas as pl
from jax.experimental.pallas import tpu as pltpu
from jax.experimental.pallas import tpu_sc as plsc

NC = 2
NS = 16
L = 16
K = 40


def _make_segsum(N, E, D, with_cnt):
    NW = NC * NS
    EPW = E // NW
    NCH = EPW // K
    B0 = (N // NS) // 8 * 8
    TAIL = N - NS * B0
    ZR = 24
    assert E == NW * EPW and EPW == NCH * K
    assert NCH >= 8 and NCH % 8 >= 2
    assert B0 % ZR == 0 and TAIL % 8 == 0 and TAIL <= ZR and N % 8 == 0

    mesh = plsc.VectorSubcoreMesh(core_axis_name="c", subcore_axis_name="s")

    NB = 4
    NI = 8
    out_type = [jax.ShapeDtypeStruct((NC, N, D), jnp.float32)]
    scratch = (
        [pltpu.VMEM_SHARED((N, D), jnp.float32)]
        + [pltpu.VMEM((K,), jnp.int32) for _ in range(NI)]
        + [pltpu.VMEM((K,), jnp.int32) for _ in range(NI)]
        + [pltpu.VMEM((K, D), jnp.float32) for _ in range(NB)]
        + [pltpu.VMEM((ZR, D), jnp.float32)]
        + [pltpu.SemaphoreType.DMA for _ in range(NI)]
        + [pltpu.SemaphoreType.DMA for _ in range(NI)]
        + [pltpu.SemaphoreType.DMA for _ in range(NB)]
        + [pltpu.SemaphoreType.DMA for _ in range(NB)]
        + [pltpu.SemaphoreType.DMA]
    )
    if with_cnt:
        out_type.append(jax.ShapeDtypeStruct((NC * N,), jnp.float32))
        scratch = list(scratch) + (
            [pltpu.VMEM_SHARED((N,), jnp.float32)]
            + [pltpu.VMEM((K,), jnp.float32)]
            + [pltpu.VMEM((B0,), jnp.float32)]
            + [pltpu.SemaphoreType.DMA for _ in range(NB)]
        )

    @functools.partial(pl.kernel, out_type=tuple(out_type), mesh=mesh,
                       scratch_types=scratch)
    def seg(*refs):
        if with_cnt:
            x_hbm, src_hbm, dst_hbm, agg_out, cnt_out = refs[:5]
            r = list(refs[5:])
        else:
            x_hbm, src_hbm, dst_hbm, agg_out = refs[:4]
            r = list(refs[4:])
        agg_sh = r.pop(0)
        ss8 = [r.pop(0) for _ in range(NI)]
        dd8 = [r.pop(0) for _ in range(NI)]
        bufs = [r.pop(0) for _ in range(NB)]
        z128 = r.pop(0)
        sis = [r.pop(0) for _ in range(NI)]
        sds = [r.pop(0) for _ in range(NI)]
        sgs = [r.pop(0) for _ in range(NB)]
        ssc = [r.pop(0) for _ in range(NB)]
        szr = r.pop(0)
        if with_cnt:
            cnt_sh = r.pop(0)
            ones_k = r.pop(0)
            z1d = r.pop(0)
            scn = [r.pop(0) for _ in range(NB)]

        cid = lax.axis_index("c")
        sid = lax.axis_index("s")
        wid = sid * NC + cid

        def fire_idx(g, j):
            pltpu.async_copy(src_hbm.at[wid, g, 0], ss8[j], sis[j])
            pltpu.async_copy(dst_hbm.at[wid, g, 0], dd8[j], sds[j])

        def wait_idx(g, j):
            pltpu.make_async_copy(src_hbm.at[wid, g, 0], ss8[j], sis[j]).wait()
            pltpu.make_async_copy(dst_hbm.at[wid, g, 0], dd8[j], sds[j]).wait()

        def fire_gather(j, b):
            pltpu.async_copy(x_hbm.at[ss8[j]], bufs[b], sgs[b])

        def wait_gather(j, b):
            pltpu.make_async_copy(x_hbm.at[ss8[j]], bufs[b], sgs[b]).wait()

        def fire_scatter(j, b):
            pltpu.async_copy(bufs[b], agg_sh.at[dd8[j]], ssc[b], add=True)

        def wait_scatter(b):
            pltpu.make_async_copy(bufs[b], agg_sh.at[dd8[0]], ssc[b]).wait()

        def fire_cnt(j, b):
            pltpu.async_copy(ones_k, cnt_sh.at[dd8[j]], scn[b], add=True)

        def wait_cnt(b):
            pltpu.make_async_copy(ones_k, cnt_sh.at[dd8[0]], scn[b]).wait()

        for g0 in range(6):
            fire_idx(g0, g0 % NI)

        zv = jnp.zeros((L,), jnp.float32)

        def fill_z128(i, c):
            for j in range(D // L):
                z128[i, pl.ds(j * L, L)] = zv
            return c
        lax.fori_loop(0, ZR, fill_z128, 0)
        if with_cnt:
            ov = jnp.ones((L,), jnp.float32)
            for j in range(K // L):
                ones_k[pl.ds(j * L, L)] = ov

            def fill_z1d(i, c):
                z1d[pl.ds(i * L, L)] = zv
                return c
            lax.fori_loop(0, B0 // L, fill_z1d, 0)

        zcopies = [(z128, agg_sh.at[pl.ds(sid * B0 + j * ZR, ZR)])
                   for j in range(B0 // ZR)]
        if with_cnt:
            zcopies.append((z1d, cnt_sh.at[pl.ds(sid * B0, B0)]))
        for zsrc, zdst in zcopies:
            pltpu.async_copy(zsrc, zdst, szr)

        @pl.when(sid == NS - 1)
        def _ztail():
            pltpu.sync_copy(z128.at[pl.ds(0, TAIL)],
                            agg_sh.at[pl.ds(NS * B0, TAIL)])
            if with_cnt:
                pltpu.sync_copy(z1d.at[pl.ds(0, TAIL)],
                                cnt_sh.at[pl.ds(NS * B0, TAIL)])

        wait_idx(0, 0)
        fire_gather(0, 0)
        wait_idx(1, 1)
        fire_gather(1, 1)

        for zsrc, zdst in zcopies:
            pltpu.make_async_copy(zsrc, zdst, szr).wait()
        plsc.subcore_barrier()

        def body(i, c):
            for b in range(NI):
                g = NI * i + b
                db_ = b % NB
                wait_gather(b, db_)
                fire_scatter(b, db_)
                if with_cnt:
                    @pl.when(g >= NB)
                    def _wc():
                        wait_cnt(db_)
                    fire_cnt(b, db_)

                @pl.when(g >= 2)
                def _ws():
                    wait_scatter((b + 2) % NB)

                @pl.when(g + 6 < NCH)
                def _fi():
                    fire_idx(g + 6, (b + 6) % NI)
                wait_idx(g + 2, (b + 2) % NI)
                fire_gather((b + 2) % NI, (b + 2) % NB)
            return c
        lax.fori_loop(0, NCH // NI, body, 0)

        for g in range(NCH // NI * NI, NCH):
            b = g % NI
            db_ = g % NB
            wait_gather(b, db_)
            fire_scatter(b, db_)
            if with_cnt:
                if g >= NB:
                    wait_cnt(db_)
                fire_cnt(b, db_)
            if g >= 2:
                wait_scatter((g + 2) % NB)
            if g + 6 < NCH:
                fire_idx(g + 6, (g + 6) % NI)
            if g + 2 < NCH:
                wait_idx(g + 2, (g + 2) % NI)
                fire_gather((g + 2) % NI, (g + 2) % NB)

        wait_scatter((NCH - 2) % NB)
        wait_scatter((NCH - 1) % NB)
        if with_cnt:
            for g in range(max(0, NCH - NB), NCH):
                wait_cnt(g % NB)

        plsc.subcore_barrier()

        pltpu.sync_copy(agg_sh.at[pl.ds(sid * B0, B0)],
                        agg_out.at[cid, pl.ds(sid * B0, B0)])
        if with_cnt:
            pltpu.sync_copy(cnt_sh.at[pl.ds(sid * B0, B0)], z1d)
            pltpu.sync_copy(z1d, cnt_out.at[pl.ds(cid * N + sid * B0, B0)])

        @pl.when(sid == NS - 1)
        def _ctail():
            pltpu.sync_copy(agg_sh.at[pl.ds(NS * B0, TAIL)],
                            agg_out.at[cid, pl.ds(NS * B0, TAIL)])
            if with_cnt:
                pltpu.sync_copy(cnt_sh.at[pl.ds(NS * B0, TAIL)],
                                z1d.at[pl.ds(0, TAIL)])
                pltpu.sync_copy(z1d.at[pl.ds(0, TAIL)],
                                cnt_out.at[pl.ds(cid * N + NS * B0, TAIL)])

    return seg


def _tca_body(x_ref, wr_ref, b_ref, o_ref):
    o_ref[...] = (jnp.dot(x_ref[...], wr_ref[...],
                          preferred_element_type=jnp.float32)
                  + b_ref[...])


def _tc1b_body(xr_ref, s_ref, c_ref, wn_ref, o_ref):
    s = s_ref[0] + s_ref[1]
    deg = c_ref[:, 0:1] + c_ref[:, 1:2]
    mean = s / jnp.maximum(deg, 1.0)
    o_ref[...] = xr_ref[...] + jnp.dot(mean, wn_ref[...],
                                       preferred_element_type=jnp.float32)


def _tcc_body(x1_ref, wr_ref, b_ref, lwa_ref, xr_ref, xl_ref):
    x1 = x1_ref[...]
    xr_ref[...] = (jnp.dot(x1, wr_ref[...],
                           preferred_element_type=jnp.float32) + b_ref[...])
    xl_ref[...] = jnp.dot(x1, lwa_ref[...],
                          preferred_element_type=jnp.float32)


def _tc2b_body(xr_ref, xl_ref, s_ref, c_ref, wn_ref, g_ref, be_ref,
               lwb_ref, lb_ref, o_ref):
    s = s_ref[0] + s_ref[1]
    deg = c_ref[:, 0:1] + c_ref[:, 1:2]
    mean = s / jnp.maximum(deg, 1.0)
    mu = jnp.mean(mean, axis=-1, keepdims=True)
    var = jnp.mean((mean - mu) * (mean - mu), axis=-1, keepdims=True)
    mean = (mean - mu) / jnp.sqrt(var + 1e-5)
    mean = mean * g_ref[...] + be_ref[...]
    x2 = xr_ref[...] + jnp.dot(mean, wn_ref[...],
                               preferred_element_type=jnp.float32)
    out = (xl_ref[...]
           + jnp.dot(x2, lwb_ref[...], preferred_element_type=jnp.float32)
           + lb_ref[...])
    o_ref[...] = jnp.maximum(out, 0.0)


def _row_spec(R, D):
    return pl.BlockSpec((R, D), lambda i: (i, 0))


def _pair_spec(R, D):
    return pl.BlockSpec((2, R, D), lambda i: (0, i, 0))


def _full_spec(a, b):
    return pl.BlockSpec((a, b), lambda i: (0, 0))


@jax.jit
def kernel(x, edge_index, W_root1, W_neigh1, b1, W_root2, W_neigh2, b2,
           gamma, beta, lin_W, lin_b):
    N, D = x.shape
    E = edge_index.shape[1]
    NW = NC * NS
    NCH = E // (NW * K)
    src = edge_index[0].reshape(NW, NCH, 1, K)
    dst = edge_index[1].reshape(NW, NCH, 1, K)

    seg1 = _make_segsum(N, E, D, with_cnt=True)
    seg2 = _make_segsum(N, E, D, with_cnt=False)

    sums1, cnt_flat = seg1(x, src, dst)
    deg2 = cnt_flat.reshape(NC, N).T

    R = 400
    G = N // R
    f32 = jnp.float32

    xr1 = pl.pallas_call(
        _tca_body,
        grid=(G,),
        in_specs=[_row_spec(R, D), _full_spec(D, D), _full_spec(1, D)],
        out_specs=_row_spec(R, D),
        out_shape=jax.ShapeDtypeStruct((N, D), f32),
    )(x, W_root1, b1.reshape(1, D))

    x1 = pl.pallas_call(
        _tc1b_body,
        grid=(G,),
        in_specs=[_row_spec(R, D), _pair_spec(R, D), _row_spec(R, NC),
                  _full_spec(D, D)],
        out_specs=_row_spec(R, D),
        out_shape=jax.ShapeDtypeStruct((N, D), f32),
    )(xr1, sums1, deg2, W_neigh1)

    (sums2,) = seg2(x1, src, dst)

    xr2, xl1 = pl.pallas_call(
        _tcc_body,
        grid=(G,),
        in_specs=[_row_spec(R, D), _full_spec(D, D), _full_spec(1, D),
                  _full_spec(D, D)],
        out_specs=[_row_spec(R, D), _row_spec(R, D)],
        out_shape=[jax.ShapeDtypeStruct((N, D), f32),
                   jax.ShapeDtypeStruct((N, D), f32)],
    )(x1, W_root2, b2.reshape(1, D), lin_W[:, :D].T)

    out = pl.pallas_call(
        _tc2b_body,
        grid=(G,),
        in_specs=[_row_spec(R, D), _row_spec(R, D), _pair_spec(R, D),
                  _row_spec(R, NC), _full_spec(D, D), _full_spec(1, D),
                  _full_spec(1, D), _full_spec(D, D), _full_spec(1, D)],
        out_specs=_row_spec(R, D),
        out_shape=jax.ShapeDtypeStruct((N, D), f32),
    )(xr2, xl1, sums2, deg2, W_neigh2, gamma.reshape(1, D),
      beta.reshape(1, D), lin_W[:, D:].T, lin_b.reshape(1, D))

    return out


# speedup vs baseline: 1.1907x; 1.1907x over previous
import functools

import jax
import jax.numpy as jnp
from jax import lax
from jax.experimental import pallas as pl
from jax.experimental.pallas import tpu as pltpu
from jax.experimental.pallas import tpu_sc as plsc

NC = 2
NS = 16
L = 16
K = 80


def _make_segsum(N, E, D, with_cnt):
    NW = NC * NS
    EPW = E // NW
    NCH = EPW // K
    B0 = (N // NS) // 8 * 8
    TAIL = N - NS * B0
    ZR = 24
    assert E == NW * EPW and EPW == NCH * K
    assert NCH >= 8 and NCH % 8 >= 2
    assert B0 % ZR == 0 and TAIL % 8 == 0 and TAIL <= ZR and N % 8 == 0

    mesh = plsc.VectorSubcoreMesh(core_axis_name="c", subcore_axis_name="s")

    NB = 4
    NI = 8
    out_type = [jax.ShapeDtypeStruct((NC, N, D), jnp.float32)]
    scratch = (
        [pltpu.VMEM_SHARED((N, D), jnp.float32)]
        + [pltpu.VMEM((K,), jnp.int32) for _ in range(NI)]
        + [pltpu.VMEM((K,), jnp.int32) for _ in range(NI)]
        + [pltpu.VMEM((K, D), jnp.float32) for _ in range(NB)]
        + [pltpu.VMEM((ZR, D), jnp.float32)]
        + [pltpu.SemaphoreType.DMA for _ in range(NI)]
        + [pltpu.SemaphoreType.DMA for _ in range(NI)]
        + [pltpu.SemaphoreType.DMA for _ in range(NB)]
        + [pltpu.SemaphoreType.DMA for _ in range(NB)]
        + [pltpu.SemaphoreType.DMA]
    )
    if with_cnt:
        out_type.append(jax.ShapeDtypeStruct((NC * N,), jnp.float32))
        scratch = list(scratch) + (
            [pltpu.VMEM_SHARED((N,), jnp.float32)]
            + [pltpu.VMEM((K,), jnp.float32)]
            + [pltpu.VMEM((B0,), jnp.float32)]
            + [pltpu.SemaphoreType.DMA for _ in range(NB)]
        )

    @functools.partial(pl.kernel, out_type=tuple(out_type), mesh=mesh,
                       scratch_types=scratch)
    def seg(*refs):
        if with_cnt:
            x_hbm, src_hbm, dst_hbm, agg_out, cnt_out = refs[:5]
            r = list(refs[5:])
        else:
            x_hbm, src_hbm, dst_hbm, agg_out = refs[:4]
            r = list(refs[4:])
        agg_sh = r.pop(0)
        ss8 = [r.pop(0) for _ in range(NI)]
        dd8 = [r.pop(0) for _ in range(NI)]
        bufs = [r.pop(0) for _ in range(NB)]
        z128 = r.pop(0)
        sis = [r.pop(0) for _ in range(NI)]
        sds = [r.pop(0) for _ in range(NI)]
        sgs = [r.pop(0) for _ in range(NB)]
        ssc = [r.pop(0) for _ in range(NB)]
        szr = r.pop(0)
        if with_cnt:
            cnt_sh = r.pop(0)
            ones_k = r.pop(0)
            z1d = r.pop(0)
            scn = [r.pop(0) for _ in range(NB)]

        cid = lax.axis_index("c")
        sid = lax.axis_index("s")
        wid = sid * NC + cid

        def fire_idx(g, j):
            pltpu.async_copy(src_hbm.at[wid, g, 0], ss8[j], sis[j])
            pltpu.async_copy(dst_hbm.at[wid, g, 0], dd8[j], sds[j])

        def wait_idx(g, j):
            pltpu.make_async_copy(src_hbm.at[wid, g, 0], ss8[j], sis[j]).wait()
            pltpu.make_async_copy(dst_hbm.at[wid, g, 0], dd8[j], sds[j]).wait()

        def fire_gather(j, b):
            pltpu.async_copy(x_hbm.at[ss8[j]], bufs[b], sgs[b])

        def wait_gather(j, b):
            pltpu.make_async_copy(x_hbm.at[ss8[j]], bufs[b], sgs[b]).wait()

        def fire_scatter(j, b):
            pltpu.async_copy(bufs[b], agg_sh.at[dd8[j]], ssc[b], add=True)

        def wait_scatter(b):
            pltpu.make_async_copy(bufs[b], agg_sh.at[dd8[0]], ssc[b]).wait()

        def fire_cnt(j, b):
            pltpu.async_copy(ones_k, cnt_sh.at[dd8[j]], scn[b], add=True)

        def wait_cnt(b):
            pltpu.make_async_copy(ones_k, cnt_sh.at[dd8[0]], scn[b]).wait()

        for g0 in range(6):
            fire_idx(g0, g0 % NI)

        zv = jnp.zeros((L,), jnp.float32)

        def fill_z128(i, c):
            for j in range(D // L):
                z128[i, pl.ds(j * L, L)] = zv
            return c
        lax.fori_loop(0, ZR, fill_z128, 0)
        if with_cnt:
            ov = jnp.ones((L,), jnp.float32)
            for j in range(K // L):
                ones_k[pl.ds(j * L, L)] = ov

            def fill_z1d(i, c):
                z1d[pl.ds(i * L, L)] = zv
                return c
            lax.fori_loop(0, B0 // L, fill_z1d, 0)

        zcopies = [(z128, agg_sh.at[pl.ds(sid * B0 + j * ZR, ZR)])
                   for j in range(B0 // ZR)]
        if with_cnt:
            zcopies.append((z1d, cnt_sh.at[pl.ds(sid * B0, B0)]))
        for zsrc, zdst in zcopies:
            pltpu.async_copy(zsrc, zdst, szr)

        @pl.when(sid == NS - 1)
        def _ztail():
            pltpu.sync_copy(z128.at[pl.ds(0, TAIL)],
                            agg_sh.at[pl.ds(NS * B0, TAIL)])
            if with_cnt:
                pltpu.sync_copy(z1d.at[pl.ds(0, TAIL)],
                                cnt_sh.at[pl.ds(NS * B0, TAIL)])

        wait_idx(0, 0)
        fire_gather(0, 0)
        wait_idx(1, 1)
        fire_gather(1, 1)

        for zsrc, zdst in zcopies:
            pltpu.make_async_copy(zsrc, zdst, szr).wait()
        plsc.subcore_barrier()

        def body(i, c):
            for b in range(NI):
                g = NI * i + b
                db_ = b % NB
                wait_gather(b, db_)
                fire_scatter(b, db_)
                if with_cnt:
                    @pl.when(g >= NB)
                    def _wc():
                        wait_cnt(db_)
                    fire_cnt(b, db_)

                @pl.when(g >= 2)
                def _ws():
                    wait_scatter((b + 2) % NB)

                @pl.when(g + 6 < NCH)
                def _fi():
                    fire_idx(g + 6, (b + 6) % NI)
                wait_idx(g + 2, (b + 2) % NI)
                fire_gather((b + 2) % NI, (b + 2) % NB)
            return c
        lax.fori_loop(0, NCH // NI, body, 0)

        for g in range(NCH // NI * NI, NCH):
            b = g % NI
            db_ = g % NB
            wait_gather(b, db_)
            fire_scatter(b, db_)
            if with_cnt:
                if g >= NB:
                    wait_cnt(db_)
                fire_cnt(b, db_)
            if g >= 2:
                wait_scatter((g + 2) % NB)
            if g + 6 < NCH:
                fire_idx(g + 6, (g + 6) % NI)
            if g + 2 < NCH:
                wait_idx(g + 2, (g + 2) % NI)
                fire_gather((g + 2) % NI, (g + 2) % NB)

        wait_scatter((NCH - 2) % NB)
        wait_scatter((NCH - 1) % NB)
        if with_cnt:
            for g in range(max(0, NCH - NB), NCH):
                wait_cnt(g % NB)

        plsc.subcore_barrier()

        pltpu.sync_copy(agg_sh.at[pl.ds(sid * B0, B0)],
                        agg_out.at[cid, pl.ds(sid * B0, B0)])
        if with_cnt:
            pltpu.sync_copy(cnt_sh.at[pl.ds(sid * B0, B0)], z1d)
            pltpu.sync_copy(z1d, cnt_out.at[pl.ds(cid * N + sid * B0, B0)])

        @pl.when(sid == NS - 1)
        def _ctail():
            pltpu.sync_copy(agg_sh.at[pl.ds(NS * B0, TAIL)],
                            agg_out.at[cid, pl.ds(NS * B0, TAIL)])
            if with_cnt:
                pltpu.sync_copy(cnt_sh.at[pl.ds(NS * B0, TAIL)],
                                z1d.at[pl.ds(0, TAIL)])
                pltpu.sync_copy(z1d.at[pl.ds(0, TAIL)],
                                cnt_out.at[pl.ds(cid * N + NS * B0, TAIL)])

    return seg


def _tca_body(x_ref, wr_ref, b_ref, o_ref):
    o_ref[...] = (jnp.dot(x_ref[...], wr_ref[...],
                          preferred_element_type=jnp.float32)
                  + b_ref[...])


def _tc1b_body(xr_ref, s_ref, c_ref, wn_ref, o_ref):
    s = s_ref[0] + s_ref[1]
    deg = c_ref[:, 0:1] + c_ref[:, 1:2]
    mean = s / jnp.maximum(deg, 1.0)
    o_ref[...] = xr_ref[...] + jnp.dot(mean, wn_ref[...],
                                       preferred_element_type=jnp.float32)


def _tcc_body(x1_ref, wr_ref, b_ref, lwa_ref, xr_ref, xl_ref):
    x1 = x1_ref[...]
    xr_ref[...] = (jnp.dot(x1, wr_ref[...],
                           preferred_element_type=jnp.float32) + b_ref[...])
    xl_ref[...] = jnp.dot(x1, lwa_ref[...],
                          preferred_element_type=jnp.float32)


def _tc2b_body(xr_ref, xl_ref, s_ref, c_ref, wn_ref, g_ref, be_ref,
               lwb_ref, lb_ref, o_ref):
    s = s_ref[0] + s_ref[1]
    deg = c_ref[:, 0:1] + c_ref[:, 1:2]
    mean = s / jnp.maximum(deg, 1.0)
    mu = jnp.mean(mean, axis=-1, keepdims=True)
    var = jnp.mean((mean - mu) * (mean - mu), axis=-1, keepdims=True)
    mean = (mean - mu) / jnp.sqrt(var + 1e-5)
    mean = mean * g_ref[...] + be_ref[...]
    x2 = xr_ref[...] + jnp.dot(mean, wn_ref[...],
                               preferred_element_type=jnp.float32)
    out = (xl_ref[...]
           + jnp.dot(x2, lwb_ref[...], preferred_element_type=jnp.float32)
           + lb_ref[...])
    o_ref[...] = jnp.maximum(out, 0.0)


def _row_spec(R, D):
    return pl.BlockSpec((R, D), lambda i: (i, 0))


def _pair_spec(R, D):
    return pl.BlockSpec((2, R, D), lambda i: (0, i, 0))


def _full_spec(a, b):
    return pl.BlockSpec((a, b), lambda i: (0, 0))


@jax.jit
def kernel(x, edge_index, W_root1, W_neigh1, b1, W_root2, W_neigh2, b2,
           gamma, beta, lin_W, lin_b):
    N, D = x.shape
    E = edge_index.shape[1]
    NW = NC * NS
    NCH = E // (NW * K)
    src = edge_index[0].reshape(NW, NCH, 1, K)
    dst = edge_index[1].reshape(NW, NCH, 1, K)

    seg1 = _make_segsum(N, E, D, with_cnt=True)
    seg2 = _make_segsum(N, E, D, with_cnt=False)

    sums1, cnt_flat = seg1(x, src, dst)
    deg2 = cnt_flat.reshape(NC, N).T

    R = 400
    G = N // R
    f32 = jnp.float32

    xr1 = pl.pallas_call(
        _tca_body,
        grid=(G,),
        in_specs=[_row_spec(R, D), _full_spec(D, D), _full_spec(1, D)],
        out_specs=_row_spec(R, D),
        out_shape=jax.ShapeDtypeStruct((N, D), f32),
    )(x, W_root1, b1.reshape(1, D))

    x1 = pl.pallas_call(
        _tc1b_body,
        grid=(G,),
        in_specs=[_row_spec(R, D), _pair_spec(R, D), _row_spec(R, NC),
                  _full_spec(D, D)],
        out_specs=_row_spec(R, D),
        out_shape=jax.ShapeDtypeStruct((N, D), f32),
    )(xr1, sums1, deg2, W_neigh1)

    (sums2,) = seg2(x1, src, dst)

    xr2, xl1 = pl.pallas_call(
        _tcc_body,
        grid=(G,),
        in_specs=[_row_spec(R, D), _full_spec(D, D), _full_spec(1, D),
                  _full_spec(D, D)],
        out_specs=[_row_spec(R, D), _row_spec(R, D)],
        out_shape=[jax.ShapeDtypeStruct((N, D), f32),
                   jax.ShapeDtypeStruct((N, D), f32)],
    )(x1, W_root2, b2.reshape(1, D), lin_W[:, :D].T)

    out = pl.pallas_call(
        _tc2b_body,
        grid=(G,),
        in_specs=[_row_spec(R, D), _row_spec(R, D), _pair_spec(R, D),
                  _row_spec(R, NC), _full_spec(D, D), _full_spec(1, D),
                  _full_spec(1, D), _full_spec(D, D), _full_spec(1, D)],
        out_specs=_row_spec(R, D),
        out_shape=jax.ShapeDtypeStruct((N, D), f32),
    )(xr2, xl1, sums2, deg2, W_neigh2, gamma.reshape(1, D),
      beta.reshape(1, D), lin_W[:, D:].T, lin_b.reshape(1, D))

    return out
